# baseline (device time: 18045 ns/iter reference)
import jax
import jax.numpy as jnp
from jax import lax
from jax.experimental import pallas as pl
from jax.experimental.pallas import tpu as pltpu


def kernel(x, pi):
    def body(pi_ref, x_ref, out_ref, send_sem, recv_sem):
        my_x = lax.axis_index("x")
        my_y = lax.axis_index("y")
        my_z = lax.axis_index("z")
        dst_x = pi_ref[my_x]

        barrier_sem = pltpu.get_barrier_semaphore()
        pl.semaphore_signal(
            barrier_sem,
            inc=1,
            device_id=(dst_x, my_y, my_z),
            device_id_type=pl.DeviceIdType.MESH,
        )
        pl.semaphore_wait(barrier_sem, 1)

        rdma = pltpu.make_async_remote_copy(
            src_ref=x_ref,
            dst_ref=out_ref,
            send_sem=send_sem,
            recv_sem=recv_sem,
            device_id=(dst_x, my_y, my_z),
            device_id_type=pl.DeviceIdType.MESH,
        )
        rdma.start()
        rdma.wait()

    return pl.pallas_call(
        body,
        out_shape=jax.ShapeDtypeStruct(x.shape, x.dtype),
        in_specs=[
            pl.BlockSpec(memory_space=pltpu.SMEM),
            pl.BlockSpec(memory_space=pltpu.VMEM),
        ],
        out_specs=pl.BlockSpec(memory_space=pltpu.VMEM),
        scratch_shapes=[
            pltpu.SemaphoreType.DMA,
            pltpu.SemaphoreType.DMA,
        ],
        compiler_params=pltpu.CompilerParams(collective_id=0),
    )(pi, x)


# device time: 18012 ns/iter; 1.0018x vs baseline; 1.0018x over previous
import jax
import jax.numpy as jnp
from jax import lax
from jax.experimental import pallas as pl
from jax.experimental.pallas import tpu as pltpu


def kernel(x, pi):
    def body(pi_ref, x_ref, out_ref, send_sem, recv_sem):
        my_x = lax.axis_index("x")
        my_y = lax.axis_index("y")
        my_z = lax.axis_index("z")
        dst_x = pi_ref[my_x]

        barrier_sem = pltpu.get_barrier_semaphore()
        pl.semaphore_signal(
            barrier_sem,
            inc=1,
            device_id=(dst_x, my_y, my_z),
            device_id_type=pl.DeviceIdType.MESH,
        )
        pl.semaphore_wait(barrier_sem, 1)

        rdma = pltpu.make_async_remote_copy(
            src_ref=x_ref,
            dst_ref=out_ref,
            send_sem=send_sem,
            recv_sem=recv_sem,
            device_id=(dst_x, my_y, my_z),
            device_id_type=pl.DeviceIdType.MESH,
        )
        rdma.start()
        rdma.wait()

    return pl.pallas_call(
        body,
        out_shape=jax.ShapeDtypeStruct(x.shape, x.dtype),
        in_specs=[
            pl.BlockSpec(memory_space=pltpu.SMEM),
            pl.BlockSpec(memory_space=pl.ANY),
        ],
        out_specs=pl.BlockSpec(memory_space=pl.ANY),
        scratch_shapes=[
            pltpu.SemaphoreType.DMA,
            pltpu.SemaphoreType.DMA,
        ],
        compiler_params=pltpu.CompilerParams(collective_id=0),
    )(pi, x)


# device time: 16333 ns/iter; 1.1048x vs baseline; 1.1028x over previous
import jax
import jax.numpy as jnp
from jax import lax
from jax.experimental import pallas as pl
from jax.experimental.pallas import tpu as pltpu

H = 256
C = 4
R = H // C


def kernel(x, pi):
    def body(pi_ref, x_ref, out_ref,
             x_send_sems, x_recv_sems, y_send_sems, y_recv_sems):
        my_x = lax.axis_index("x")
        my_y = lax.axis_index("y")
        my_z = lax.axis_index("z")
        dst_x = pi_ref[my_x]
        other_y = 1 - my_y

        base_me = my_y * H
        base_other = other_y * H

        barrier_sem = pltpu.get_barrier_semaphore()
        pl.semaphore_signal(
            barrier_sem, inc=1,
            device_id=(dst_x, my_y, my_z),
            device_id_type=pl.DeviceIdType.MESH,
        )
        pl.semaphore_signal(
            barrier_sem, inc=1,
            device_id=(my_x, other_y, my_z),
            device_id_type=pl.DeviceIdType.MESH,
        )
        pl.semaphore_wait(barrier_sem, 2)

        x_rdmas = []
        for k in range(C):
            rows = pl.ds(base_me + k * R, R)
            rdma = pltpu.make_async_remote_copy(
                src_ref=x_ref.at[0, rows, :],
                dst_ref=out_ref.at[0, rows, :],
                send_sem=x_send_sems.at[k],
                recv_sem=x_recv_sems.at[k],
                device_id=(dst_x, my_y, my_z),
                device_id_type=pl.DeviceIdType.MESH,
            )
            rdma.start()
            x_rdmas.append(rdma)

        y_rdmas = []
        for k in range(C):
            x_rdmas[k].wait_recv()
            rows = pl.ds(base_me + k * R, R)
            fwd = pltpu.make_async_remote_copy(
                src_ref=out_ref.at[0, rows, :],
                dst_ref=out_ref.at[0, rows, :],
                send_sem=y_send_sems.at[k],
                recv_sem=y_recv_sems.at[k],
                device_id=(my_x, other_y, my_z),
                device_id_type=pl.DeviceIdType.MESH,
            )
            fwd.start()
            y_rdmas.append(fwd)

        for k in range(C):
            rows = pl.ds(base_other + k * R, R)
            recv = pltpu.make_async_remote_copy(
                src_ref=out_ref.at[0, rows, :],
                dst_ref=out_ref.at[0, rows, :],
                send_sem=y_send_sems.at[k],
                recv_sem=y_recv_sems.at[k],
                device_id=(my_x, other_y, my_z),
                device_id_type=pl.DeviceIdType.MESH,
            )
            recv.wait_recv()

        for k in range(C):
            x_rdmas[k].wait_send()
            y_rdmas[k].wait_send()

    return pl.pallas_call(
        body,
        out_shape=jax.ShapeDtypeStruct(x.shape, x.dtype),
        in_specs=[
            pl.BlockSpec(memory_space=pltpu.SMEM),
            pl.BlockSpec(memory_space=pl.ANY),
        ],
        out_specs=pl.BlockSpec(memory_space=pl.ANY),
        scratch_shapes=[
            pltpu.SemaphoreType.DMA((C,)),
            pltpu.SemaphoreType.DMA((C,)),
            pltpu.SemaphoreType.DMA((C,)),
            pltpu.SemaphoreType.DMA((C,)),
        ],
        compiler_params=pltpu.CompilerParams(collective_id=0),
    )(pi, x)


# device time: 15780 ns/iter; 1.1435x vs baseline; 1.0350x over previous
import jax
import jax.numpy as jnp
from jax import lax
from jax.experimental import pallas as pl
from jax.experimental.pallas import tpu as pltpu

H = 256
C = 8
R = H // C


def kernel(x, pi):
    def body(pi_ref, x_ref, out_ref,
             x_send_sems, x_recv_sems, y_send_sems, y_recv_sems):
        my_x = lax.axis_index("x")
        my_y = lax.axis_index("y")
        my_z = lax.axis_index("z")
        dst_x = pi_ref[my_x]
        other_y = 1 - my_y

        base_me = my_y * H
        base_other = other_y * H

        barrier_sem = pltpu.get_barrier_semaphore()
        pl.semaphore_signal(
            barrier_sem, inc=1,
            device_id=(dst_x, my_y, my_z),
            device_id_type=pl.DeviceIdType.MESH,
        )
        pl.semaphore_signal(
            barrier_sem, inc=1,
            device_id=(my_x, other_y, my_z),
            device_id_type=pl.DeviceIdType.MESH,
        )
        pl.semaphore_wait(barrier_sem, 2)

        x_rdmas = []
        for k in range(C):
            rows = pl.ds(base_me + k * R, R)
            rdma = pltpu.make_async_remote_copy(
                src_ref=x_ref.at[0, rows, :],
                dst_ref=out_ref.at[0, rows, :],
                send_sem=x_send_sems.at[k],
                recv_sem=x_recv_sems.at[k],
                device_id=(dst_x, my_y, my_z),
                device_id_type=pl.DeviceIdType.MESH,
            )
            rdma.start()
            x_rdmas.append(rdma)

        y_rdmas = []
        for k in range(C):
            x_rdmas[k].wait_recv()
            rows = pl.ds(base_me + k * R, R)
            fwd = pltpu.make_async_remote_copy(
                src_ref=out_ref.at[0, rows, :],
                dst_ref=out_ref.at[0, rows, :],
                send_sem=y_send_sems.at[k],
                recv_sem=y_recv_sems.at[k],
                device_id=(my_x, other_y, my_z),
                device_id_type=pl.DeviceIdType.MESH,
            )
            fwd.start()
            y_rdmas.append(fwd)

        for k in range(C):
            rows = pl.ds(base_other + k * R, R)
            recv = pltpu.make_async_remote_copy(
                src_ref=out_ref.at[0, rows, :],
                dst_ref=out_ref.at[0, rows, :],
                send_sem=y_send_sems.at[k],
                recv_sem=y_recv_sems.at[k],
                device_id=(my_x, other_y, my_z),
                device_id_type=pl.DeviceIdType.MESH,
            )
            recv.wait_recv()

        for k in range(C):
            x_rdmas[k].wait_send()
            y_rdmas[k].wait_send()

    return pl.pallas_call(
        body,
        out_shape=jax.ShapeDtypeStruct(x.shape, x.dtype),
        in_specs=[
            pl.BlockSpec(memory_space=pltpu.SMEM),
            pl.BlockSpec(memory_space=pl.ANY),
        ],
        out_specs=pl.BlockSpec(memory_space=pl.ANY),
        scratch_shapes=[
            pltpu.SemaphoreType.DMA((C,)),
            pltpu.SemaphoreType.DMA((C,)),
            pltpu.SemaphoreType.DMA((C,)),
            pltpu.SemaphoreType.DMA((C,)),
        ],
        compiler_params=pltpu.CompilerParams(collective_id=0),
    )(pi, x)
